# baseline (speedup 1.0000x reference)
import functools

import jax
import jax.numpy as jnp
from jax import lax
from jax.experimental import pallas as pl
from jax.experimental.pallas import tpu as pltpu
from jax.experimental.pallas import tpu_sc as plsc

TOPK = 32
CHUNK = 128
NEG_INF = float("-inf")
NC, NS, NL = 2, 16, 16
NWORK = NC * NS


def _enc_body(x_ref, bpre_ref, w_ref, benc_ref, pre_ref, cm_ref):
    xc = (x_ref[...] - bpre_ref[...]).astype(jnp.bfloat16)
    acc = lax.dot_general(
        xc, w_ref[...], (((1,), (1,)), ((), ())),
        preferred_element_type=jnp.float32)
    pre = acc + benc_ref[...]
    pre_ref[...] = pre
    nchunk = pre.shape[1] // CHUNK
    cm_ref[0] = jnp.concatenate(
        [jnp.max(pre[:, c * CHUNK:(c + 1) * CHUNK], axis=1, keepdims=True)
         for c in range(nchunk)], axis=1)


def _encoder(x, b_pre, W_enc, b_enc, bn, bl):
    N, D = x.shape
    L = W_enc.shape[0]
    grid = (N // bn, L // bl)
    return pl.pallas_call(
        _enc_body,
        grid=grid,
        in_specs=[
            pl.BlockSpec((bn, D), lambda i, j: (i, 0)),
            pl.BlockSpec((1, D), lambda i, j: (0, 0)),
            pl.BlockSpec((bl, D), lambda i, j: (j, 0)),
            pl.BlockSpec((1, bl), lambda i, j: (0, j)),
        ],
        out_specs=[
            pl.BlockSpec((bn, bl), lambda i, j: (i, j)),
            pl.BlockSpec((1, bn, bl // CHUNK), lambda i, j: (j, i, 0)),
        ],
        out_shape=[
            jax.ShapeDtypeStruct((N, L), jnp.float32),
            jax.ShapeDtypeStruct((L // bl, N, bl // CHUNK), jnp.float32),
        ],
    )(x, b_pre.reshape(1, D), W_enc.astype(jnp.bfloat16),
      b_enc.reshape(1, L))


def _ctop_body(cm_ref, cid_ref, thr_ref, k):
    nblk, bn, w = cm_ref.shape
    nch = nblk * w
    cm = jnp.concatenate([cm_ref[c] for c in range(nblk)], axis=1)
    lane = lax.broadcasted_iota(jnp.int32, (bn, nch), 1)
    klane = lax.broadcasted_iota(jnp.int32, (bn, k), 1)

    def step(i, carry):
        work, acc = carry
        m = jnp.max(work, axis=1, keepdims=True)
        cand = jnp.where(work == m, lane, jnp.int32(nch))
        am = jnp.min(cand, axis=1, keepdims=True)
        work = jnp.where(lane == am, NEG_INF, work)
        acc = jnp.where(klane == i, am, acc)
        thr_ref[...] = m
        return (work, acc)

    _, acc = lax.fori_loop(0, k, step, (cm, jnp.zeros((bn, k), jnp.int32)))

    def sstep(i, carry):
        work, out = carry
        mn = jnp.min(work, axis=1, keepdims=True)
        out = jnp.where(klane == i, mn, out)
        work = jnp.where(work == mn, jnp.int32(nch), work)
        return (work, out)

    _, acc = lax.fori_loop(0, k, sstep, (acc, acc))
    rowbase = (lax.broadcasted_iota(jnp.int32, (bn, k), 0)
               + pl.program_id(0) * bn) * nch
    cid_ref[...] = acc + rowbase


def _chunk_topk(cmax3, bn, k):
    nblk, N, w = cmax3.shape
    nch = nblk * w
    return pl.pallas_call(
        functools.partial(_ctop_body, k=k),
        grid=(N // bn,),
        in_specs=[pl.BlockSpec((nblk, bn, w), lambda i: (0, i, 0))],
        out_specs=[
            pl.BlockSpec((bn, k), lambda i: (i, 0)),
            pl.BlockSpec((bn, 1), lambda i: (i, 0)),
        ],
        out_shape=[
            jax.ShapeDtypeStruct((N, k), jnp.int32),
            jax.ShapeDtypeStruct((N, 1), jnp.float32),
        ],
    )(cmax3)


def _splat_i32(v):
    return jnp.full((NL,), v, jnp.int32)


def _sc_make(N, L, D, NCH_ROW):
    RPW = N // NWORK
    NCHSEL = TOPK
    CB = NCHSEL * CHUNK
    ND = D // NL
    mesh = plsc.VectorSubcoreMesh(
        core_axis_name="c", subcore_axis_name="s",
        num_cores=NC, num_subcores=NS)

    TV = TOPK + NL

    def body(pre_hbm, cids_hbm, thr_hbm, wdecT_hbm, bias_hbm,
             z_hbm, xhat_hbm,
             cids_v, thrv, cb2d, cv, ci, zbuf, wbuf, accbuf, idxbuf,
             tpos, tvalf, biasv, sem_cb, sem_z, sem_x, sem_w):
        wid = lax.axis_index("s") * NC + lax.axis_index("c")
        base = wid * RPW
        pltpu.sync_copy(cids_hbm.at[pl.ds(base, RPW)], cids_v)
        pltpu.sync_copy(thr_hbm.at[pl.ds(base, RPW + NL)], thrv)
        pltpu.sync_copy(bias_hbm, biasv)

        lane = lax.iota(jnp.int32, NL)
        lane0 = lane == 0
        zeros16 = jnp.zeros((NL,), jnp.float32)

        def zinit(i, c):
            zbuf[pl.ds(i * NL, NL)] = zeros16
            return c
        lax.fori_loop(0, 2 * L // NL, zinit, 0)

        pltpu.async_copy(pre_hbm.at[cids_v.at[0]], cb2d.at[0], sem_cb.at[0])
        pltpu.async_copy(pre_hbm.at[cids_v.at[1]], cb2d.at[1], sem_cb.at[1])

        def decode_row(parity, rowid):
            pltpu.make_async_copy(wdecT_hbm.at[idxbuf.at[parity]],
                                  wbuf.at[parity], sem_w.at[parity]).wait()
            ob = parity * TV
            acc0 = tuple(biasv[pl.ds(d * NL, NL)] for d in range(ND))

            def dec(j, acc):
                scale = jnp.maximum(tvalf[pl.ds(ob + j, NL)][0], 0.0)
                return tuple(acc[d] + wbuf[parity, j, pl.ds(d * NL, NL)] * scale
                             for d in range(ND))
            acc = lax.fori_loop(0, TOPK, dec, acc0)
            for d in range(ND):
                accbuf[parity, pl.ds(d * NL, NL)] = acc[d]
            pltpu.async_copy(accbuf.at[parity], xhat_hbm.at[rowid],
                             sem_x.at[parity])

        def row_body(r, c):
            row = base + r
            b = r & 1
            pltpu.make_async_copy(pre_hbm.at[cids_v.at[r]], cb2d.at[b],
                                  sem_cb.at[b]).wait()
            thr_s = thrv[pl.ds(r, NL)][0]

            @pl.when(r >= 2)
            def _():
                pltpu.make_async_copy(zbuf.at[pl.ds(b * L, L)],
                                      z_hbm.at[row - 2], sem_z.at[b]).wait()
                pltpu.make_async_copy(accbuf.at[b], xhat_hbm.at[row - 2],
                                      sem_x.at[b]).wait()
                og0 = idxbuf[b, pl.ds(0, NL)]
                og1 = idxbuf[b, pl.ds(NL, NL)]
                plsc.store_scatter(zbuf, [og0 + b * L], zeros16)
                plsc.store_scatter(zbuf, [og1 + b * L], zeros16)

            def filt(i, cnt):
                jj = i // (CHUNK // NL)
                kk = (i % (CHUNK // NL)) * NL
                v = cb2d[b, jj, pl.ds(kk, NL)]
                m = v >= thr_s
                nhit = plsc.all_reduce_population_count(m)[0]

                @pl.when(nhit > 0)
                def _():
                    gi = lane + i * NL
                    pos = cnt + plsc.cumsum(jnp.where(m, 1, 0)) - 1
                    plsc.store_scatter(cv, [pos], v, mask=m)
                    plsc.store_scatter(ci, [pos], gi, mask=m)
                return cnt + nhit
            cnt = lax.fori_loop(0, CB // NL, filt, jnp.int32(0))

            @pl.when(r + 2 < RPW)
            def _():
                pltpu.async_copy(pre_hbm.at[cids_v.at[r + 2]], cb2d.at[b],
                                 sem_cb.at[b])
            cv[pl.ds(cnt, NL)] = jnp.full((NL,), NEG_INF, jnp.float32)
            nv = (cnt + NL - 1) // NL
            ob = b * TV

            def sel(i, c2):
                def scan_v(jv, best):
                    bm, bj = best
                    v = cv[pl.ds(jv * NL, NL)]
                    lm = jnp.max(v)
                    better = lm > bm
                    return (jnp.where(better, lm, bm),
                            jnp.where(better, jv, bj))
                bm, bj = lax.fori_loop(0, nv, scan_v,
                                       (jnp.float32(NEG_INF), jnp.int32(0)))
                v = cv[pl.ds(bj * NL, NL)]
                lane_hit = plsc.all_reduce_ffs(v == bm)[0]
                pos = bj * NL + lane_hit
                plsc.store_scatter(cv, [_splat_i32(pos)],
                                   jnp.full((NL,), NEG_INF, jnp.float32),
                                   mask=lane0)
                plsc.store_scatter(tpos, [_splat_i32(i)], _splat_i32(pos),
                                   mask=lane0)
                plsc.store_scatter(tvalf, [_splat_i32(ob + i)],
                                   jnp.full((NL,), bm, jnp.float32),
                                   mask=lane0)
                return c2
            lax.fori_loop(0, TOPK, sel, 0)

            p0 = tpos[pl.ds(0, NL)]
            p1 = tpos[pl.ds(NL, NL)]
            lp0 = plsc.load_gather(ci, [p0])
            lp1 = plsc.load_gather(ci, [p1])
            cs0 = lax.shift_right_logical(lp0, 7)
            cs1 = lax.shift_right_logical(lp1, 7)
            g0 = plsc.load_gather(cids_v, [_splat_i32(r), cs0])
            g1 = plsc.load_gather(cids_v, [_splat_i32(r), cs1])
            gi0 = g0 * CHUNK - row * L + (lp0 & (CHUNK - 1))
            gi1 = g1 * CHUNK - row * L + (lp1 & (CHUNK - 1))
            rv0 = jnp.maximum(tvalf[pl.ds(ob, NL)], 0.0)
            rv1 = jnp.maximum(tvalf[pl.ds(ob + NL, NL)], 0.0)

            idxbuf[b, pl.ds(0, NL)] = gi0
            idxbuf[b, pl.ds(NL, NL)] = gi1
            pltpu.async_copy(wdecT_hbm.at[idxbuf.at[b]], wbuf.at[b],
                             sem_w.at[b])

            plsc.store_scatter(zbuf, [gi0 + b * L], rv0)
            plsc.store_scatter(zbuf, [gi1 + b * L], rv1)
            pltpu.async_copy(zbuf.at[pl.ds(b * L, L)], z_hbm.at[row],
                             sem_z.at[b])

            @pl.when(r >= 1)
            def _():
                decode_row(1 - b, row - 1)
            return c
        lax.fori_loop(0, RPW, row_body, 0)

        decode_row((RPW - 1) & 1, base + RPW - 1)

        for b in range(2):
            pltpu.make_async_copy(zbuf.at[pl.ds(b * L, L)],
                                  z_hbm.at[base + RPW - 2 + b],
                                  sem_z.at[b]).wait()
            pltpu.make_async_copy(accbuf.at[b], xhat_hbm.at[base + RPW - 2 + b],
                                  sem_x.at[b]).wait()

    return functools.partial(
        pl.kernel, body, mesh=mesh,
        compiler_params=pltpu.CompilerParams(needs_layout_passes=False),
        out_type=(jax.ShapeDtypeStruct((N, L), jnp.float32),
                  jax.ShapeDtypeStruct((N, D), jnp.float32)),
        scratch_types=[
            pltpu.VMEM((RPW, TOPK), jnp.int32),
            pltpu.VMEM((RPW + NL,), jnp.float32),
            pltpu.VMEM((2, NCHSEL, CHUNK), jnp.float32),
            pltpu.VMEM((CB + NL,), jnp.float32),
            pltpu.VMEM((CB + NL,), jnp.int32),
            pltpu.VMEM((2 * L,), jnp.float32),
            pltpu.VMEM((2, TOPK, D), jnp.float32),
            pltpu.VMEM((2, D), jnp.float32),
            pltpu.VMEM((2, TOPK), jnp.int32),
            pltpu.VMEM((TOPK,), jnp.int32),
            pltpu.VMEM((2 * (TOPK + NL),), jnp.float32),
            pltpu.VMEM((D,), jnp.float32),
            pltpu.SemaphoreType.DMA((2,)),
            pltpu.SemaphoreType.DMA((2,)),
            pltpu.SemaphoreType.DMA((2,)),
            pltpu.SemaphoreType.DMA((2,)),
        ])()


def kernel(x, b_pre, W_enc, b_enc, W_dec, b_dec):
    N, D = x.shape
    L = W_enc.shape[0]
    pre, cmax3 = _encoder(x, b_pre, W_enc, b_enc, min(256, N), min(2048, L))
    cids, thr = _chunk_topk(cmax3, min(256, N), TOPK)
    thr_pad = jnp.pad(thr.reshape(N), (0, NL))
    bias = b_dec + b_pre
    wdecT = W_dec.T.reshape(L, D)
    sc = _sc_make(N, L, D, L // CHUNK)
    z, x_hat = sc(pre.reshape(N * (L // CHUNK), CHUNK), cids, thr_pad,
                  wdecT, bias)
    return (pre, z, x_hat)


# speedup vs baseline: 1.1976x; 1.1976x over previous
import functools

import jax
import jax.numpy as jnp
from jax import lax
from jax.experimental import pallas as pl
from jax.experimental.pallas import tpu as pltpu
from jax.experimental.pallas import tpu_sc as plsc

TOPK = 32
CHUNK = 128
NEG_INF = float("-inf")
NC, NS, NL = 2, 16, 16
NWORK = NC * NS


def _enc_body(x_ref, bpre_ref, w_ref, benc_ref, pre_ref, cm_ref):
    xc = (x_ref[...] - bpre_ref[...]).astype(jnp.bfloat16)
    acc = lax.dot_general(
        xc, w_ref[...], (((1,), (1,)), ((), ())),
        preferred_element_type=jnp.float32)
    pre = acc + benc_ref[...]
    pre_ref[...] = pre
    nchunk = pre.shape[1] // CHUNK
    cm_ref[0] = jnp.concatenate(
        [jnp.max(pre[:, c * CHUNK:(c + 1) * CHUNK], axis=1, keepdims=True)
         for c in range(nchunk)], axis=1)


def _encoder(x, b_pre, W_enc, b_enc, bn, bl):
    N, D = x.shape
    L = W_enc.shape[0]
    grid = (L // bl, N // bn)
    return pl.pallas_call(
        _enc_body,
        grid=grid,
        in_specs=[
            pl.BlockSpec((bn, D), lambda j, i: (i, 0)),
            pl.BlockSpec((1, D), lambda j, i: (0, 0)),
            pl.BlockSpec((bl, D), lambda j, i: (j, 0)),
            pl.BlockSpec((1, bl), lambda j, i: (0, j)),
        ],
        out_specs=[
            pl.BlockSpec((bn, bl), lambda j, i: (i, j)),
            pl.BlockSpec((1, bn, bl // CHUNK), lambda j, i: (j, i, 0)),
        ],
        out_shape=[
            jax.ShapeDtypeStruct((N, L), jnp.float32),
            jax.ShapeDtypeStruct((L // bl, N, bl // CHUNK), jnp.float32),
        ],
    )(x, b_pre.reshape(1, D), W_enc.astype(jnp.bfloat16),
      b_enc.reshape(1, L))


def _ctop_body(cm_ref, cid_ref, thr_ref, k):
    nblk, bn, w = cm_ref.shape
    nch = nblk * w
    cm = jnp.concatenate([cm_ref[c] for c in range(nblk)], axis=1)
    lane = lax.broadcasted_iota(jnp.int32, (bn, nch), 1)
    klane = lax.broadcasted_iota(jnp.int32, (bn, k), 1)

    def step(i, carry):
        work, acc = carry
        m = jnp.max(work, axis=1, keepdims=True)
        cand = jnp.where(work == m, lane, jnp.int32(nch))
        am = jnp.min(cand, axis=1, keepdims=True)
        work = jnp.where(lane == am, NEG_INF, work)
        acc = jnp.where(klane == i, am, acc)
        thr_ref[...] = m
        return (work, acc)

    _, acc = lax.fori_loop(0, k, step, (cm, jnp.zeros((bn, k), jnp.int32)))

    def sstep(i, carry):
        work, out = carry
        mn = jnp.min(work, axis=1, keepdims=True)
        out = jnp.where(klane == i, mn, out)
        work = jnp.where(work == mn, jnp.int32(nch), work)
        return (work, out)

    _, acc = lax.fori_loop(0, k, sstep, (acc, acc))
    rowbase = (lax.broadcasted_iota(jnp.int32, (bn, k), 0)
               + pl.program_id(0) * bn) * nch
    cid_ref[...] = acc + rowbase


def _chunk_topk(cmax3, bn, k):
    nblk, N, w = cmax3.shape
    nch = nblk * w
    return pl.pallas_call(
        functools.partial(_ctop_body, k=k),
        grid=(N // bn,),
        in_specs=[pl.BlockSpec((nblk, bn, w), lambda i: (0, i, 0))],
        out_specs=[
            pl.BlockSpec((bn, k), lambda i: (i, 0)),
            pl.BlockSpec((bn, 1), lambda i: (i, 0)),
        ],
        out_shape=[
            jax.ShapeDtypeStruct((N, k), jnp.int32),
            jax.ShapeDtypeStruct((N, 1), jnp.float32),
        ],
    )(cmax3)


def _splat_i32(v):
    return jnp.full((NL,), v, jnp.int32)


def _sc_make(N, L, D, NCH_ROW):
    RPW = N // NWORK
    NCHSEL = TOPK
    CB = NCHSEL * CHUNK
    ND = D // NL
    mesh = plsc.VectorSubcoreMesh(
        core_axis_name="c", subcore_axis_name="s",
        num_cores=NC, num_subcores=NS)

    TV = TOPK + NL

    def body(pre_hbm, cids_hbm, thr_hbm, wdecT_hbm, bias_hbm,
             z_hbm, xhat_hbm,
             cids_v, thrv, cb2d, cv, ci, zbuf, wbuf, accbuf, idxbuf,
             tpos, tvalf, biasv, sem_cb, sem_z, sem_x, sem_w):
        wid = lax.axis_index("s") * NC + lax.axis_index("c")
        base = wid * RPW
        pltpu.sync_copy(cids_hbm.at[pl.ds(base, RPW)], cids_v)
        pltpu.sync_copy(thr_hbm.at[pl.ds(base, RPW + NL)], thrv)
        pltpu.sync_copy(bias_hbm, biasv)

        lane = lax.iota(jnp.int32, NL)
        lane0 = lane == 0
        zeros16 = jnp.zeros((NL,), jnp.float32)

        def zinit(i, c):
            zbuf[pl.ds(i * NL, NL)] = zeros16
            return c
        lax.fori_loop(0, 2 * L // NL, zinit, 0)

        pltpu.async_copy(pre_hbm.at[cids_v.at[0]], cb2d.at[0], sem_cb.at[0])
        pltpu.async_copy(pre_hbm.at[cids_v.at[1]], cb2d.at[1], sem_cb.at[1])

        def decode_row(parity, rowid):
            pltpu.make_async_copy(wdecT_hbm.at[idxbuf.at[parity]],
                                  wbuf.at[parity], sem_w.at[parity]).wait()
            ob = parity * TV
            acc0 = tuple(biasv[pl.ds(d * NL, NL)] for d in range(ND))

            def dec(j, acc):
                scale = jnp.maximum(tvalf[pl.ds(ob + j, NL)][0], 0.0)
                return tuple(acc[d] + wbuf[parity, j, pl.ds(d * NL, NL)] * scale
                             for d in range(ND))
            acc = lax.fori_loop(0, TOPK, dec, acc0)
            for d in range(ND):
                accbuf[parity, pl.ds(d * NL, NL)] = acc[d]
            pltpu.async_copy(accbuf.at[parity], xhat_hbm.at[rowid],
                             sem_x.at[parity])

        def row_body(r, c):
            row = base + r
            b = r & 1
            pltpu.make_async_copy(pre_hbm.at[cids_v.at[r]], cb2d.at[b],
                                  sem_cb.at[b]).wait()
            thr_s = thrv[pl.ds(r, NL)][0]

            @pl.when(r >= 2)
            def _():
                pltpu.make_async_copy(zbuf.at[pl.ds(b * L, L)],
                                      z_hbm.at[row - 2], sem_z.at[b]).wait()
                pltpu.make_async_copy(accbuf.at[b], xhat_hbm.at[row - 2],
                                      sem_x.at[b]).wait()
                og0 = idxbuf[b, pl.ds(0, NL)]
                og1 = idxbuf[b, pl.ds(NL, NL)]
                plsc.store_scatter(zbuf, [og0 + b * L], zeros16)
                plsc.store_scatter(zbuf, [og1 + b * L], zeros16)

            def filt(i, cnt):
                jj = i // (CHUNK // NL)
                kk = (i % (CHUNK // NL)) * NL
                v = cb2d[b, jj, pl.ds(kk, NL)]
                gi = lane + i * NL
                m = v >= thr_s
                pos = cnt + plsc.cumsum(jnp.where(m, 1, 0)) - 1
                plsc.store_scatter(cv, [pos], v, mask=m)
                plsc.store_scatter(ci, [pos], gi, mask=m)
                return cnt + plsc.all_reduce_population_count(m)[0]
            cnt = lax.fori_loop(0, CB // NL, filt, jnp.int32(0))

            @pl.when(r + 2 < RPW)
            def _():
                pltpu.async_copy(pre_hbm.at[cids_v.at[r + 2]], cb2d.at[b],
                                 sem_cb.at[b])
            cv[pl.ds(cnt, NL)] = jnp.full((NL,), NEG_INF, jnp.float32)
            nv = (cnt + NL - 1) // NL
            ob = b * TV

            def sel(i, c2):
                def scan_v(jv, best):
                    bm, bj = best
                    v = cv[pl.ds(jv * NL, NL)]
                    lm = jnp.max(v)
                    better = lm > bm
                    return (jnp.where(better, lm, bm),
                            jnp.where(better, jv, bj))
                bm, bj = lax.fori_loop(0, nv, scan_v,
                                       (jnp.float32(NEG_INF), jnp.int32(0)))
                v = cv[pl.ds(bj * NL, NL)]
                lane_hit = plsc.all_reduce_ffs(v == bm)[0]
                pos = bj * NL + lane_hit
                plsc.store_scatter(cv, [_splat_i32(pos)],
                                   jnp.full((NL,), NEG_INF, jnp.float32),
                                   mask=lane0)
                plsc.store_scatter(tpos, [_splat_i32(i)], _splat_i32(pos),
                                   mask=lane0)
                plsc.store_scatter(tvalf, [_splat_i32(ob + i)],
                                   jnp.full((NL,), bm, jnp.float32),
                                   mask=lane0)
                return c2
            lax.fori_loop(0, TOPK, sel, 0)

            p0 = tpos[pl.ds(0, NL)]
            p1 = tpos[pl.ds(NL, NL)]
            lp0 = plsc.load_gather(ci, [p0])
            lp1 = plsc.load_gather(ci, [p1])
            cs0 = lax.shift_right_logical(lp0, 7)
            cs1 = lax.shift_right_logical(lp1, 7)
            g0 = plsc.load_gather(cids_v, [_splat_i32(r), cs0])
            g1 = plsc.load_gather(cids_v, [_splat_i32(r), cs1])
            gi0 = g0 * CHUNK - row * L + (lp0 & (CHUNK - 1))
            gi1 = g1 * CHUNK - row * L + (lp1 & (CHUNK - 1))
            rv0 = jnp.maximum(tvalf[pl.ds(ob, NL)], 0.0)
            rv1 = jnp.maximum(tvalf[pl.ds(ob + NL, NL)], 0.0)

            idxbuf[b, pl.ds(0, NL)] = gi0
            idxbuf[b, pl.ds(NL, NL)] = gi1
            pltpu.async_copy(wdecT_hbm.at[idxbuf.at[b]], wbuf.at[b],
                             sem_w.at[b])

            plsc.store_scatter(zbuf, [gi0 + b * L], rv0)
            plsc.store_scatter(zbuf, [gi1 + b * L], rv1)
            pltpu.async_copy(zbuf.at[pl.ds(b * L, L)], z_hbm.at[row],
                             sem_z.at[b])

            @pl.when(r >= 1)
            def _():
                decode_row(1 - b, row - 1)
            return c
        lax.fori_loop(0, RPW, row_body, 0)

        decode_row((RPW - 1) & 1, base + RPW - 1)

        for b in range(2):
            pltpu.make_async_copy(zbuf.at[pl.ds(b * L, L)],
                                  z_hbm.at[base + RPW - 2 + b],
                                  sem_z.at[b]).wait()
            pltpu.make_async_copy(accbuf.at[b], xhat_hbm.at[base + RPW - 2 + b],
                                  sem_x.at[b]).wait()

    return functools.partial(
        pl.kernel, body, mesh=mesh,
        compiler_params=pltpu.CompilerParams(needs_layout_passes=False),
        out_type=(jax.ShapeDtypeStruct((N, L), jnp.float32),
                  jax.ShapeDtypeStruct((N, D), jnp.float32)),
        scratch_types=[
            pltpu.VMEM((RPW, TOPK), jnp.int32),
            pltpu.VMEM((RPW + NL,), jnp.float32),
            pltpu.VMEM((2, NCHSEL, CHUNK), jnp.float32),
            pltpu.VMEM((CB + NL,), jnp.float32),
            pltpu.VMEM((CB + NL,), jnp.int32),
            pltpu.VMEM((2 * L,), jnp.float32),
            pltpu.VMEM((2, TOPK, D), jnp.float32),
            pltpu.VMEM((2, D), jnp.float32),
            pltpu.VMEM((2, TOPK), jnp.int32),
            pltpu.VMEM((TOPK,), jnp.int32),
            pltpu.VMEM((2 * (TOPK + NL),), jnp.float32),
            pltpu.VMEM((D,), jnp.float32),
            pltpu.SemaphoreType.DMA((2,)),
            pltpu.SemaphoreType.DMA((2,)),
            pltpu.SemaphoreType.DMA((2,)),
            pltpu.SemaphoreType.DMA((2,)),
        ])()


def kernel(x, b_pre, W_enc, b_enc, W_dec, b_dec):
    N, D = x.shape
    L = W_enc.shape[0]
    pre, cmax3 = _encoder(x, b_pre, W_enc, b_enc, min(256, N), min(2048, L))
    cids, thr = _chunk_topk(cmax3, min(256, N), TOPK)
    thr_pad = jnp.pad(thr.reshape(N), (0, NL))
    bias = b_dec + b_pre
    wdecT = W_dec.T.reshape(L, D)
    sc = _sc_make(N, L, D, L // CHUNK)
    z, x_hat = sc(pre.reshape(N * (L // CHUNK), CHUNK), cids, thr_pad,
                  wdecT, bias)
    return (pre, z, x_hat)


# speedup vs baseline: 1.2337x; 1.0301x over previous
import functools

import jax
import jax.numpy as jnp
from jax import lax
from jax.experimental import pallas as pl
from jax.experimental.pallas import tpu as pltpu
from jax.experimental.pallas import tpu_sc as plsc

TOPK = 32
CHUNK = 128
NEG_INF = float("-inf")
NC, NS, NL = 2, 16, 16
NWORK = NC * NS


def _enc_body(x_ref, bpre_ref, w_ref, benc_ref, pre_ref, cm_ref):
    xc = (x_ref[...] - bpre_ref[...]).astype(jnp.bfloat16)
    acc = lax.dot_general(
        xc, w_ref[...], (((1,), (1,)), ((), ())),
        preferred_element_type=jnp.float32)
    pre = acc + benc_ref[...]
    pre_ref[...] = pre
    nchunk = pre.shape[1] // CHUNK
    cm_ref[0] = jnp.concatenate(
        [jnp.max(pre[:, c * CHUNK:(c + 1) * CHUNK], axis=1, keepdims=True)
         for c in range(nchunk)], axis=1)


def _encoder(x, b_pre, W_enc, b_enc, bn, bl):
    N, D = x.shape
    L = W_enc.shape[0]
    grid = (L // bl, N // bn)
    return pl.pallas_call(
        _enc_body,
        grid=grid,
        in_specs=[
            pl.BlockSpec((bn, D), lambda j, i: (i, 0)),
            pl.BlockSpec((1, D), lambda j, i: (0, 0)),
            pl.BlockSpec((bl, D), lambda j, i: (j, 0)),
            pl.BlockSpec((1, bl), lambda j, i: (0, j)),
        ],
        out_specs=[
            pl.BlockSpec((bn, bl), lambda j, i: (i, j)),
            pl.BlockSpec((1, bn, bl // CHUNK), lambda j, i: (j, i, 0)),
        ],
        out_shape=[
            jax.ShapeDtypeStruct((N, L), jnp.float32),
            jax.ShapeDtypeStruct((L // bl, N, bl // CHUNK), jnp.float32),
        ],
    )(x, b_pre.reshape(1, D), W_enc.astype(jnp.bfloat16),
      b_enc.reshape(1, L))


def _ctop_body(cm_ref, cid_ref, thr_ref, k):
    nblk, bn, w = cm_ref.shape
    nch = nblk * w
    cm = jnp.concatenate([cm_ref[c] for c in range(nblk)], axis=1)
    lane = lax.broadcasted_iota(jnp.int32, (bn, nch), 1)
    klane = lax.broadcasted_iota(jnp.int32, (bn, k), 1)

    def step(i, carry):
        work, acc = carry
        m = jnp.max(work, axis=1, keepdims=True)
        cand = jnp.where(work == m, lane, jnp.int32(nch))
        am = jnp.min(cand, axis=1, keepdims=True)
        work = jnp.where(lane == am, NEG_INF, work)
        acc = jnp.where(klane == i, am, acc)
        thr_ref[...] = m
        return (work, acc)

    _, acc = lax.fori_loop(0, k, step, (cm, jnp.zeros((bn, k), jnp.int32)))

    def sstep(i, carry):
        work, out = carry
        mn = jnp.min(work, axis=1, keepdims=True)
        out = jnp.where(klane == i, mn, out)
        work = jnp.where(work == mn, jnp.int32(nch), work)
        return (work, out)

    _, acc = lax.fori_loop(0, k, sstep, (acc, acc))
    rowbase = (lax.broadcasted_iota(jnp.int32, (bn, k), 0)
               + pl.program_id(0) * bn) * nch
    cid_ref[...] = acc + rowbase


def _chunk_topk(cmax3, bn, k):
    nblk, N, w = cmax3.shape
    nch = nblk * w
    return pl.pallas_call(
        functools.partial(_ctop_body, k=k),
        grid=(N // bn,),
        in_specs=[pl.BlockSpec((nblk, bn, w), lambda i: (0, i, 0))],
        out_specs=[
            pl.BlockSpec((bn, k), lambda i: (i, 0)),
            pl.BlockSpec((bn, 1), lambda i: (i, 0)),
        ],
        out_shape=[
            jax.ShapeDtypeStruct((N, k), jnp.int32),
            jax.ShapeDtypeStruct((N, 1), jnp.float32),
        ],
    )(cmax3)


def _splat_i32(v):
    return jnp.full((NL,), v, jnp.int32)


def _sc_make(N, L, D, NCH_ROW):
    RPW = N // NWORK
    NCHSEL = TOPK
    CB = NCHSEL * CHUNK
    ND = D // NL
    mesh = plsc.VectorSubcoreMesh(
        core_axis_name="c", subcore_axis_name="s",
        num_cores=NC, num_subcores=NS)

    TV = TOPK + NL

    def body(pre_hbm, cids_hbm, thr_hbm, wdecT_hbm, bias_hbm,
             z_hbm, xhat_hbm,
             cids_v, thrv, cb2d, cv, ci, zbuf, wbuf, accbuf, idxbuf,
             tpos, tvalf, biasv, sem_cb, sem_z, sem_x, sem_w):
        wid = lax.axis_index("s") * NC + lax.axis_index("c")
        base = wid * RPW
        pltpu.sync_copy(cids_hbm.at[pl.ds(base, RPW)], cids_v)
        pltpu.sync_copy(thr_hbm.at[pl.ds(base, RPW + NL)], thrv)
        pltpu.sync_copy(bias_hbm, biasv)

        lane = lax.iota(jnp.int32, NL)
        lane0 = lane == 0
        zeros16 = jnp.zeros((NL,), jnp.float32)

        def zinit(i, c):
            zbuf[pl.ds(i * NL, NL)] = zeros16
            return c
        lax.fori_loop(0, 2 * L // NL, zinit, 0)

        pltpu.async_copy(pre_hbm.at[cids_v.at[0]], cb2d.at[0], sem_cb.at[0])
        pltpu.async_copy(pre_hbm.at[cids_v.at[1]], cb2d.at[1], sem_cb.at[1])

        def decode_row(parity, rowid):
            pltpu.make_async_copy(wdecT_hbm.at[idxbuf.at[parity]],
                                  wbuf.at[parity], sem_w.at[parity]).wait()
            ob = parity * TV
            acc0 = tuple(biasv[pl.ds(d * NL, NL)] for d in range(ND))

            def dec(j, acc):
                scale = jnp.maximum(tvalf[pl.ds(ob + j, NL)][0], 0.0)
                return tuple(acc[d] + wbuf[parity, j, pl.ds(d * NL, NL)] * scale
                             for d in range(ND))
            acc = lax.fori_loop(0, TOPK, dec, acc0)
            for d in range(ND):
                accbuf[parity, pl.ds(d * NL, NL)] = acc[d]
            pltpu.async_copy(accbuf.at[parity], xhat_hbm.at[rowid],
                             sem_x.at[parity])

        def row_body(r, c):
            row = base + r
            b = r & 1
            pltpu.make_async_copy(pre_hbm.at[cids_v.at[r]], cb2d.at[b],
                                  sem_cb.at[b]).wait()
            thr_s = thrv[pl.ds(r, NL)][0]

            @pl.when(r >= 2)
            def _():
                pltpu.make_async_copy(zbuf.at[pl.ds(b * L, L)],
                                      z_hbm.at[row - 2], sem_z.at[b]).wait()
                pltpu.make_async_copy(accbuf.at[b], xhat_hbm.at[row - 2],
                                      sem_x.at[b]).wait()
                og0 = idxbuf[b, pl.ds(0, NL)]
                og1 = idxbuf[b, pl.ds(NL, NL)]
                plsc.store_scatter(zbuf, [og0 + b * L], zeros16)
                plsc.store_scatter(zbuf, [og1 + b * L], zeros16)

            def filt(i, cnt):
                jj = i // (CHUNK // NL)
                kk = (i % (CHUNK // NL)) * NL
                v = cb2d[b, jj, pl.ds(kk, NL)]
                gi = lane + i * NL
                m = v >= thr_s
                pos = cnt + plsc.cumsum(jnp.where(m, 1, 0)) - 1
                plsc.store_scatter(cv, [pos], v, mask=m)
                plsc.store_scatter(ci, [pos], gi, mask=m)
                return cnt + plsc.all_reduce_population_count(m)[0]
            cnt = lax.fori_loop(0, CB // NL, filt, jnp.int32(0))

            @pl.when(r + 2 < RPW)
            def _():
                pltpu.async_copy(pre_hbm.at[cids_v.at[r + 2]], cb2d.at[b],
                                 sem_cb.at[b])
            cv[pl.ds(cnt, NL)] = jnp.full((NL,), NEG_INF, jnp.float32)
            nv = (cnt + NL - 1) // NL
            ob = b * TV

            def sel(i, c2):
                def scan_v(jv, best):
                    bm, bj = best
                    v = cv[pl.ds(jv * NL, NL)]
                    lm = jnp.max(v)
                    better = lm > bm
                    return (jnp.where(better, lm, bm),
                            jnp.where(better, jv, bj))
                bm, bj = lax.fori_loop(0, nv, scan_v,
                                       (jnp.float32(NEG_INF), jnp.int32(0)))
                v = cv[pl.ds(bj * NL, NL)]
                lane_hit = plsc.all_reduce_ffs(v == bm)[0]
                pos = bj * NL + lane_hit
                plsc.store_scatter(cv, [_splat_i32(pos)],
                                   jnp.full((NL,), NEG_INF, jnp.float32),
                                   mask=lane0)
                plsc.store_scatter(tpos, [_splat_i32(i)], _splat_i32(pos),
                                   mask=lane0)
                plsc.store_scatter(tvalf, [_splat_i32(ob + i)],
                                   jnp.full((NL,), bm, jnp.float32),
                                   mask=lane0)
                return c2
            lax.fori_loop(0, TOPK, sel, 0)

            p0 = tpos[pl.ds(0, NL)]
            p1 = tpos[pl.ds(NL, NL)]
            lp0 = plsc.load_gather(ci, [p0])
            lp1 = plsc.load_gather(ci, [p1])
            cs0 = lax.shift_right_logical(lp0, 7)
            cs1 = lax.shift_right_logical(lp1, 7)
            g0 = plsc.load_gather(cids_v, [_splat_i32(r), cs0])
            g1 = plsc.load_gather(cids_v, [_splat_i32(r), cs1])
            gi0 = g0 * CHUNK - row * L + (lp0 & (CHUNK - 1))
            gi1 = g1 * CHUNK - row * L + (lp1 & (CHUNK - 1))
            rv0 = jnp.maximum(tvalf[pl.ds(ob, NL)], 0.0)
            rv1 = jnp.maximum(tvalf[pl.ds(ob + NL, NL)], 0.0)

            idxbuf[b, pl.ds(0, NL)] = gi0
            idxbuf[b, pl.ds(NL, NL)] = gi1
            pltpu.async_copy(wdecT_hbm.at[idxbuf.at[b]], wbuf.at[b],
                             sem_w.at[b])

            plsc.store_scatter(zbuf, [gi0 + b * L], rv0)
            plsc.store_scatter(zbuf, [gi1 + b * L], rv1)
            pltpu.async_copy(zbuf.at[pl.ds(b * L, L)], z_hbm.at[row],
                             sem_z.at[b])

            @pl.when(r >= 1)
            def _():
                decode_row(1 - b, row - 1)
            return c
        lax.fori_loop(0, RPW, row_body, 0)

        decode_row((RPW - 1) & 1, base + RPW - 1)

        for b in range(2):
            pltpu.make_async_copy(zbuf.at[pl.ds(b * L, L)],
                                  z_hbm.at[base + RPW - 2 + b],
                                  sem_z.at[b]).wait()
            pltpu.make_async_copy(accbuf.at[b], xhat_hbm.at[base + RPW - 2 + b],
                                  sem_x.at[b]).wait()

    return functools.partial(
        pl.kernel, body, mesh=mesh,
        compiler_params=pltpu.CompilerParams(needs_layout_passes=False),
        out_type=(jax.ShapeDtypeStruct((N, L), jnp.float32),
                  jax.ShapeDtypeStruct((N, D), jnp.float32)),
        scratch_types=[
            pltpu.VMEM((RPW, TOPK), jnp.int32),
            pltpu.VMEM((RPW + NL,), jnp.float32),
            pltpu.VMEM((2, NCHSEL, CHUNK), jnp.float32),
            pltpu.VMEM((CB + NL,), jnp.float32),
            pltpu.VMEM((CB + NL,), jnp.int32),
            pltpu.VMEM((2 * L,), jnp.float32),
            pltpu.VMEM((2, TOPK, D), jnp.float32),
            pltpu.VMEM((2, D), jnp.float32),
            pltpu.VMEM((2, TOPK), jnp.int32),
            pltpu.VMEM((TOPK,), jnp.int32),
            pltpu.VMEM((2 * (TOPK + NL),), jnp.float32),
            pltpu.VMEM((D,), jnp.float32),
            pltpu.SemaphoreType.DMA((2,)),
            pltpu.SemaphoreType.DMA((2,)),
            pltpu.SemaphoreType.DMA((2,)),
            pltpu.SemaphoreType.DMA((2,)),
        ])()


def kernel(x, b_pre, W_enc, b_enc, W_dec, b_dec):
    N, D = x.shape
    L = W_enc.shape[0]
    pre, cmax3 = _encoder(x, b_pre, W_enc, b_enc, min(256, N), min(2048, L))
    cids, thr = _chunk_topk(cmax3, min(512, N), TOPK)
    thr_pad = jnp.pad(thr.reshape(N), (0, NL))
    bias = b_dec + b_pre
    wdecT = W_dec.T.reshape(L, D)
    sc = _sc_make(N, L, D, L // CHUNK)
    z, x_hat = sc(pre.reshape(N * (L // CHUNK), CHUNK), cids, thr_pad,
                  wdecT, bias)
    return (pre, z, x_hat)


# speedup vs baseline: 1.2486x; 1.0121x over previous
import functools

import jax
import jax.numpy as jnp
from jax import lax
from jax.experimental import pallas as pl
from jax.experimental.pallas import tpu as pltpu
from jax.experimental.pallas import tpu_sc as plsc

TOPK = 32
CHUNK = 128
NEG_INF = float("-inf")
NC, NS, NL = 2, 16, 16
NWORK = NC * NS


def _enc_body(x_ref, bpre_ref, w_ref, benc_ref, pre_ref, cm_ref):
    xc = (x_ref[...] - bpre_ref[...]).astype(jnp.bfloat16)
    acc = lax.dot_general(
        xc, w_ref[...], (((1,), (1,)), ((), ())),
        preferred_element_type=jnp.float32)
    pre = acc + benc_ref[...]
    pre_ref[...] = pre
    nchunk = pre.shape[1] // CHUNK
    cm_ref[0] = jnp.concatenate(
        [jnp.max(pre[:, c * CHUNK:(c + 1) * CHUNK], axis=1, keepdims=True)
         for c in range(nchunk)], axis=1)


def _encoder(x, b_pre, W_enc, b_enc, bn, bl):
    N, D = x.shape
    L = W_enc.shape[0]
    grid = (L // bl, N // bn)
    return pl.pallas_call(
        _enc_body,
        grid=grid,
        in_specs=[
            pl.BlockSpec((bn, D), lambda j, i: (i, 0)),
            pl.BlockSpec((1, D), lambda j, i: (0, 0)),
            pl.BlockSpec((bl, D), lambda j, i: (j, 0)),
            pl.BlockSpec((1, bl), lambda j, i: (0, j)),
        ],
        out_specs=[
            pl.BlockSpec((bn, bl), lambda j, i: (i, j)),
            pl.BlockSpec((1, bn, bl // CHUNK), lambda j, i: (j, i, 0)),
        ],
        out_shape=[
            jax.ShapeDtypeStruct((N, L), jnp.float32),
            jax.ShapeDtypeStruct((L // bl, N, bl // CHUNK), jnp.float32),
        ],
    )(x, b_pre.reshape(1, D), W_enc.astype(jnp.bfloat16),
      b_enc.reshape(1, L))


def _ctop_body(cm_ref, cid_ref, thr_ref, k):
    nblk, bn, w = cm_ref.shape
    nch = nblk * w
    cm = jnp.concatenate([cm_ref[c] for c in range(nblk)], axis=1)
    lane = lax.broadcasted_iota(jnp.int32, (bn, nch), 1)
    klane = lax.broadcasted_iota(jnp.int32, (bn, k), 1)

    def step(i, carry):
        work, acc = carry
        m = jnp.max(work, axis=1, keepdims=True)
        cand = jnp.where(work == m, lane, jnp.int32(nch))
        am = jnp.min(cand, axis=1, keepdims=True)
        work = jnp.where(lane == am, NEG_INF, work)
        acc = jnp.where(klane == i, am, acc)
        thr_ref[...] = m
        return (work, acc)

    _, acc = lax.fori_loop(0, k, step, (cm, jnp.zeros((bn, k), jnp.int32)))

    def sstep(i, carry):
        work, out = carry
        mn = jnp.min(work, axis=1, keepdims=True)
        out = jnp.where(klane == i, mn, out)
        work = jnp.where(work == mn, jnp.int32(nch), work)
        return (work, out)

    _, acc = lax.fori_loop(0, k, sstep, (acc, acc))
    rowbase = (lax.broadcasted_iota(jnp.int32, (bn, k), 0)
               + pl.program_id(0) * bn) * nch
    cid_ref[...] = acc + rowbase


def _chunk_topk(cmax3, bn, k):
    nblk, N, w = cmax3.shape
    nch = nblk * w
    return pl.pallas_call(
        functools.partial(_ctop_body, k=k),
        grid=(N // bn,),
        in_specs=[pl.BlockSpec((nblk, bn, w), lambda i: (0, i, 0))],
        out_specs=[
            pl.BlockSpec((bn, k), lambda i: (i, 0)),
            pl.BlockSpec((bn, 1), lambda i: (i, 0)),
        ],
        out_shape=[
            jax.ShapeDtypeStruct((N, k), jnp.int32),
            jax.ShapeDtypeStruct((N, 1), jnp.float32),
        ],
    )(cmax3)


def _splat_i32(v):
    return jnp.full((NL,), v, jnp.int32)


def _sc_make(N, L, D, NCH_ROW):
    RPW = N // NWORK
    NCHSEL = TOPK
    CB = NCHSEL * CHUNK
    ND = D // NL
    mesh = plsc.VectorSubcoreMesh(
        core_axis_name="c", subcore_axis_name="s",
        num_cores=NC, num_subcores=NS)

    TV = TOPK + NL

    def body(pre_hbm, cids_hbm, thr_hbm, wdecT_hbm, bias_hbm,
             z_hbm, xhat_hbm,
             cids_v, thrv, cb2d, cv, ci, zbuf, wbuf, accbuf, idxbuf,
             tpos, tvalf, biasv, sem_cb, sem_z, sem_x, sem_w):
        wid = lax.axis_index("s") * NC + lax.axis_index("c")
        base = wid * RPW
        pltpu.sync_copy(cids_hbm.at[pl.ds(base, RPW)], cids_v)
        pltpu.sync_copy(thr_hbm.at[pl.ds(base, RPW + NL)], thrv)
        pltpu.sync_copy(bias_hbm, biasv)

        lane = lax.iota(jnp.int32, NL)
        lane0 = lane == 0
        zeros16 = jnp.zeros((NL,), jnp.float32)

        def zinit(i, c):
            zbuf[pl.ds(i * NL, NL)] = zeros16
            return c
        lax.fori_loop(0, 2 * L // NL, zinit, 0)

        pltpu.async_copy(pre_hbm.at[cids_v.at[0]], cb2d.at[0], sem_cb.at[0])
        pltpu.async_copy(pre_hbm.at[cids_v.at[1]], cb2d.at[1], sem_cb.at[1])

        def decode_row(parity, rowid):
            pltpu.make_async_copy(wdecT_hbm.at[idxbuf.at[parity]],
                                  wbuf.at[parity], sem_w.at[parity]).wait()
            ob = parity * TV
            acc0 = tuple(biasv[pl.ds(d * NL, NL)] for d in range(ND))

            def dec(j, acc):
                scale = jnp.maximum(tvalf[pl.ds(ob + j, NL)][0], 0.0)
                return tuple(acc[d] + wbuf[parity, j, pl.ds(d * NL, NL)] * scale
                             for d in range(ND))
            acc = lax.fori_loop(0, TOPK, dec, acc0)
            for d in range(ND):
                accbuf[parity, pl.ds(d * NL, NL)] = acc[d]
            pltpu.async_copy(accbuf.at[parity], xhat_hbm.at[rowid],
                             sem_x.at[parity])

        def row_body(r, c):
            row = base + r
            b = r & 1
            pltpu.make_async_copy(pre_hbm.at[cids_v.at[r]], cb2d.at[b],
                                  sem_cb.at[b]).wait()
            thr_s = thrv[pl.ds(r, NL)][0]

            @pl.when(r >= 2)
            def _():
                pltpu.make_async_copy(zbuf.at[pl.ds(b * L, L)],
                                      z_hbm.at[row - 2], sem_z.at[b]).wait()
                pltpu.make_async_copy(accbuf.at[b], xhat_hbm.at[row - 2],
                                      sem_x.at[b]).wait()
                og0 = idxbuf[b, pl.ds(0, NL)]
                og1 = idxbuf[b, pl.ds(NL, NL)]
                plsc.store_scatter(zbuf, [og0 + b * L], zeros16)
                plsc.store_scatter(zbuf, [og1 + b * L], zeros16)

            def filt(i, cnt):
                jj = i // (CHUNK // NL)
                kk = (i % (CHUNK // NL)) * NL
                v = cb2d[b, jj, pl.ds(kk, NL)]
                gi = lane + i * NL
                m = v >= thr_s
                pos = cnt + plsc.cumsum(jnp.where(m, 1, 0)) - 1
                plsc.store_scatter(cv, [pos], v, mask=m)
                plsc.store_scatter(ci, [pos], gi, mask=m)
                return cnt + plsc.all_reduce_population_count(m)[0]
            cnt = lax.fori_loop(0, CB // NL, filt, jnp.int32(0))

            @pl.when(r + 2 < RPW)
            def _():
                pltpu.async_copy(pre_hbm.at[cids_v.at[r + 2]], cb2d.at[b],
                                 sem_cb.at[b])
            cv[pl.ds(cnt, NL)] = jnp.full((NL,), NEG_INF, jnp.float32)
            nv = (cnt + NL - 1) // NL
            ob = b * TV

            def sel(i, c2):
                def scan_v(jv, best):
                    bm, bj = best
                    v = cv[pl.ds(jv * NL, NL)]
                    lm = jnp.max(v)
                    better = lm > bm
                    return (jnp.where(better, lm, bm),
                            jnp.where(better, jv, bj))
                bm, bj = lax.fori_loop(0, nv, scan_v,
                                       (jnp.float32(NEG_INF), jnp.int32(0)))
                v = cv[pl.ds(bj * NL, NL)]
                lane_hit = plsc.all_reduce_ffs(v == bm)[0]
                pos = bj * NL + lane_hit
                plsc.store_scatter(cv, [_splat_i32(pos)],
                                   jnp.full((NL,), NEG_INF, jnp.float32),
                                   mask=lane0)
                plsc.store_scatter(tpos, [_splat_i32(i)], _splat_i32(pos),
                                   mask=lane0)
                plsc.store_scatter(tvalf, [_splat_i32(ob + i)],
                                   jnp.full((NL,), bm, jnp.float32),
                                   mask=lane0)
                return c2
            lax.fori_loop(0, TOPK, sel, 0)

            p0 = tpos[pl.ds(0, NL)]
            p1 = tpos[pl.ds(NL, NL)]
            lp0 = plsc.load_gather(ci, [p0])
            lp1 = plsc.load_gather(ci, [p1])
            cs0 = lax.shift_right_logical(lp0, 7)
            cs1 = lax.shift_right_logical(lp1, 7)
            g0 = plsc.load_gather(cids_v, [_splat_i32(r), cs0])
            g1 = plsc.load_gather(cids_v, [_splat_i32(r), cs1])
            gi0 = g0 * CHUNK - row * L + (lp0 & (CHUNK - 1))
            gi1 = g1 * CHUNK - row * L + (lp1 & (CHUNK - 1))
            rv0 = jnp.maximum(tvalf[pl.ds(ob, NL)], 0.0)
            rv1 = jnp.maximum(tvalf[pl.ds(ob + NL, NL)], 0.0)

            idxbuf[b, pl.ds(0, NL)] = gi0
            idxbuf[b, pl.ds(NL, NL)] = gi1
            pltpu.async_copy(wdecT_hbm.at[idxbuf.at[b]], wbuf.at[b],
                             sem_w.at[b])

            plsc.store_scatter(zbuf, [gi0 + b * L], rv0)
            plsc.store_scatter(zbuf, [gi1 + b * L], rv1)
            pltpu.async_copy(zbuf.at[pl.ds(b * L, L)], z_hbm.at[row],
                             sem_z.at[b])

            @pl.when(r >= 1)
            def _():
                decode_row(1 - b, row - 1)
            return c
        lax.fori_loop(0, RPW, row_body, 0)

        decode_row((RPW - 1) & 1, base + RPW - 1)

        for b in range(2):
            pltpu.make_async_copy(zbuf.at[pl.ds(b * L, L)],
                                  z_hbm.at[base + RPW - 2 + b],
                                  sem_z.at[b]).wait()
            pltpu.make_async_copy(accbuf.at[b], xhat_hbm.at[base + RPW - 2 + b],
                                  sem_x.at[b]).wait()

    return functools.partial(
        pl.kernel, body, mesh=mesh,
        compiler_params=pltpu.CompilerParams(needs_layout_passes=False),
        out_type=(jax.ShapeDtypeStruct((N, L), jnp.float32),
                  jax.ShapeDtypeStruct((N, D), jnp.float32)),
        scratch_types=[
            pltpu.VMEM((RPW, TOPK), jnp.int32),
            pltpu.VMEM((RPW + NL,), jnp.float32),
            pltpu.VMEM((2, NCHSEL, CHUNK), jnp.float32),
            pltpu.VMEM((CB + NL,), jnp.float32),
            pltpu.VMEM((CB + NL,), jnp.int32),
            pltpu.VMEM((2 * L,), jnp.float32),
            pltpu.VMEM((2, TOPK, D), jnp.float32),
            pltpu.VMEM((2, D), jnp.float32),
            pltpu.VMEM((2, TOPK), jnp.int32),
            pltpu.VMEM((TOPK,), jnp.int32),
            pltpu.VMEM((2 * (TOPK + NL),), jnp.float32),
            pltpu.VMEM((D,), jnp.float32),
            pltpu.SemaphoreType.DMA((2,)),
            pltpu.SemaphoreType.DMA((2,)),
            pltpu.SemaphoreType.DMA((2,)),
            pltpu.SemaphoreType.DMA((2,)),
        ])()


def kernel(x, b_pre, W_enc, b_enc, W_dec, b_dec):
    N, D = x.shape
    L = W_enc.shape[0]
    pre, cmax3 = _encoder(x, b_pre, W_enc, b_enc, min(256, N), min(2048, L))
    cids, thr = _chunk_topk(cmax3, min(1024, N), TOPK)
    thr_pad = jnp.pad(thr.reshape(N), (0, NL))
    bias = b_dec + b_pre
    wdecT = W_dec.T.reshape(L, D)
    sc = _sc_make(N, L, D, L // CHUNK)
    z, x_hat = sc(pre.reshape(N * (L // CHUNK), CHUNK), cids, thr_pad,
                  wdecT, bias)
    return (pre, z, x_hat)
